# baseline (speedup 1.0000x reference)
import functools

import numpy as np
import jax
import jax.numpy as jnp
from jax.experimental import pallas as pl
from jax.experimental.pallas import tpu as pltpu

_TAU = 1.0
_B = 4096
_NNEG = 128
_P = 4
_D = 128
_R = 512
_LOG2E = 1.4426950408889634


@functools.lru_cache(maxsize=1)
def _neg_mask():
    rng = np.random.default_rng(0)
    all_idx = np.arange(_B)
    mask = np.zeros((_B, _B), dtype=np.int8)
    for i in range(_B):
        sel = rng.choice(np.delete(all_idx, i), _NNEG, replace=False)
        mask[i, sel] = 1
    return jnp.asarray(mask)


@functools.lru_cache(maxsize=1)
def _seg_w():
    w = np.zeros((_P * _D, _P), dtype=np.float32)
    for g in range(_P):
        w[g * _D:(g + 1) * _D, g] = 1.0
    return jnp.asarray(w)


def _loss_kernel(z_ref, pz_ref, mask_ref, w_ref, out_ref, zn_ref, znl_ref,
                 acc_ref):
    i = pl.program_id(0)

    @pl.when(i == 0)
    def _init():
        z = z_ref[...]
        n2 = jnp.sum(z * z, axis=1, keepdims=True)
        zn = z * jax.lax.rsqrt(jnp.maximum(n2, 1e-24))
        zn_ref[...] = zn
        znl_ref[...] = (zn * _LOG2E).astype(jnp.bfloat16)
        acc_ref[0, 0] = 0.0

    zn_blk = zn_ref[pl.ds(i * _R, _R), :]
    s2 = jax.lax.dot_general(
        zn_blk.astype(jnp.bfloat16),
        znl_ref[...],
        (((1,), (1,)), ((), ())),
        preferred_element_type=jnp.float32,
    )
    neg_e = jnp.sum(jnp.exp2(s2) * mask_ref[...].astype(jnp.float32), axis=1)

    p2 = pz_ref[...]
    zrep = jnp.concatenate([zn_blk] * _P, axis=1)
    w = w_ref[...]
    pn2 = jax.lax.dot_general(p2 * p2, w, (((1,), (0,)), ((), ())),
                              preferred_element_type=jnp.float32)
    pd = jax.lax.dot_general(p2 * zrep, w, (((1,), (0,)), ((), ())),
                             preferred_element_type=jnp.float32)
    pos_s = pd * jax.lax.rsqrt(jnp.maximum(pn2, 1e-24))
    pos_e = jnp.sum(jnp.exp(pos_s), axis=1)

    loss = jnp.log(neg_e + pos_e) - jnp.log(pos_e)
    acc_ref[0, 0] += jnp.sum(loss)

    @pl.when(i == pl.num_programs(0) - 1)
    def _finish():
        out_ref[...] = jnp.full((1, 1), acc_ref[0, 0] * (1.0 / _B), jnp.float32)


def kernel(z_vecs, pos_z_vecs):
    mask = _neg_mask()
    pz2 = jnp.reshape(pos_z_vecs, (_B, _P * _D))
    out = pl.pallas_call(
        _loss_kernel,
        grid=(_B // _R,),
        in_specs=[
            pl.BlockSpec((_B, _D), lambda i: (0, 0)),
            pl.BlockSpec((_R, _P * _D), lambda i: (i, 0)),
            pl.BlockSpec((_R, _B), lambda i: (i, 0)),
            pl.BlockSpec((_P * _D, _P), lambda i: (0, 0)),
        ],
        out_specs=pl.BlockSpec((1, 1), lambda i: (0, 0)),
        out_shape=jax.ShapeDtypeStruct((1, 1), jnp.float32),
        scratch_shapes=[
            pltpu.VMEM((_B, _D), jnp.float32),
            pltpu.VMEM((_B, _D), jnp.bfloat16),
            pltpu.SMEM((1, 1), jnp.float32),
        ],
    )(z_vecs, pz2, mask, _seg_w())
    return jnp.reshape(out, ())


# speedup vs baseline: 1.6923x; 1.6923x over previous
import functools

import numpy as np
import jax
import jax.numpy as jnp
from jax.experimental import pallas as pl
from jax.experimental.pallas import tpu as pltpu

_TAU = 1.0
_B = 4096
_NNEG = 128
_P = 4
_D = 128
_R = 512
_LOG2E = 1.4426950408889634


@functools.lru_cache(maxsize=1)
def _neg_mask():
    rng = np.random.default_rng(0)
    all_idx = np.arange(_B)
    mask = np.zeros((_B, _B), dtype=np.int8)
    for i in range(_B):
        sel = rng.choice(np.delete(all_idx, i), _NNEG, replace=False)
        mask[i, sel] = 1
    return jnp.asarray(mask)


def _loss_kernel(z_ref, pz_ref, mask_ref, out_ref, zn_ref, znl_ref,
                 acc_ref):
    i = pl.program_id(0)

    @pl.when(i == 0)
    def _init():
        z = z_ref[...]
        n2 = jnp.sum(z * z, axis=1, keepdims=True)
        zn = z * jax.lax.rsqrt(jnp.maximum(n2, 1e-24))
        zn_ref[...] = zn
        znl_ref[...] = (zn * _LOG2E).astype(jnp.bfloat16)
        acc_ref[0, 0] = 0.0

    zn_blk = zn_ref[pl.ds(i * _R, _R), :]
    s2 = jax.lax.dot_general(
        zn_blk.astype(jnp.bfloat16),
        znl_ref[...],
        (((1,), (1,)), ((), ())),
        preferred_element_type=jnp.float32,
    )
    neg_e = jnp.sum(jnp.exp2(s2) * mask_ref[...].astype(jnp.float32), axis=1)

    p3 = pz_ref[...]
    pn2 = jnp.sum(p3 * p3, axis=2)
    pd = jnp.sum(p3 * zn_blk[:, None, :], axis=2)
    pos_s = pd * jax.lax.rsqrt(jnp.maximum(pn2, 1e-24))
    pos_e = jnp.sum(jnp.exp(pos_s), axis=1)

    loss = jnp.log(neg_e + pos_e) - jnp.log(pos_e)
    acc_ref[0, 0] += jnp.sum(loss)

    @pl.when(i == pl.num_programs(0) - 1)
    def _finish():
        out_ref[...] = jnp.full((1, 1), acc_ref[0, 0] * (1.0 / _B), jnp.float32)


def kernel(z_vecs, pos_z_vecs):
    mask = _neg_mask()
    out = pl.pallas_call(
        _loss_kernel,
        grid=(_B // _R,),
        in_specs=[
            pl.BlockSpec((_B, _D), lambda i: (0, 0)),
            pl.BlockSpec((_R, _P, _D), lambda i: (i, 0, 0)),
            pl.BlockSpec((_R, _B), lambda i: (i, 0)),
        ],
        out_specs=pl.BlockSpec((1, 1), lambda i: (0, 0)),
        out_shape=jax.ShapeDtypeStruct((1, 1), jnp.float32),
        scratch_shapes=[
            pltpu.VMEM((_B, _D), jnp.float32),
            pltpu.VMEM((_B, _D), jnp.bfloat16),
            pltpu.SMEM((1, 1), jnp.float32),
        ],
    )(z_vecs, pos_z_vecs, mask)
    return jnp.reshape(out, ())
